# baseline (speedup 1.0000x reference)
import functools

import jax
import jax.numpy as jnp
from jax import lax
from jax.experimental import pallas as pl
from jax.experimental.pallas import tpu as pltpu
from jax.experimental.pallas import tpu_sc as plsc

NUM_WORKERS = 32
CHUNK = 16
NBUF = 6
AHEAD = 6


def _make_gather(nrows, ncols, embed):
    total = nrows * ncols
    per_worker = total // NUM_WORKERS
    nchunks = per_worker // CHUNK
    wpr = ncols // per_worker
    mesh = plsc.VectorSubcoreMesh(core_axis_name="c", subcore_axis_name="s")

    @functools.partial(
        pl.kernel,
        mesh=mesh,
        out_type=jax.ShapeDtypeStruct((nrows, ncols, embed), jnp.float32),
        scratch_types=[
            pltpu.VMEM((per_worker,), jnp.int32),
        ] + [pltpu.VMEM((CHUNK, embed), jnp.float32)] * NBUF
          + [pltpu.SemaphoreType.DMA] * 2,
    )
    def gather_kernel(idx_hbm, table_hbm, out_hbm, idx_v, *scratch):
        bufs = scratch[:NBUF]
        gsem, wsem = scratch[NBUF:]
        wid = lax.axis_index("c") * (NUM_WORKERS // 2) + lax.axis_index("s")
        row = wid // wpr
        col = (wid % wpr) * per_worker
        pltpu.sync_copy(idx_hbm.at[row, pl.ds(col, per_worker)], idx_v)

        def start_gather(j):
            return pltpu.async_copy(
                table_hbm.at[idx_v.at[pl.ds(j * CHUNK, CHUNK)]],
                bufs[j % NBUF], gsem)

        gds = [None] * nchunks
        wds = [None] * nchunks
        waited = [False] * nchunks
        for j in range(min(AHEAD, nchunks)):
            gds[j] = start_gather(j)
        for j in range(nchunks):
            gds[j].wait()
            wds[j] = pltpu.async_copy(
                bufs[j % NBUF],
                out_hbm.at[row, pl.ds(col + j * CHUNK, CHUNK)],
                wsem)
            nxt = j + AHEAD
            if nxt < nchunks:
                if nxt >= NBUF:
                    wds[nxt - NBUF].wait()
                    waited[nxt - NBUF] = True
                gds[nxt] = start_gather(nxt)
        for j in range(nchunks):
            if not waited[j]:
                wds[j].wait()

    return gather_kernel


def kernel(time, pos_enc):
    if time.dtype != jnp.int32:
        time = time.astype(jnp.int32)
    nrows, ncols = time.shape
    return _make_gather(nrows, ncols, pos_enc.shape[1])(time, pos_enc)


# speedup vs baseline: 1.0237x; 1.0237x over previous
import functools

import jax
import jax.numpy as jnp
from jax import lax
from jax.experimental import pallas as pl
from jax.experimental.pallas import tpu as pltpu
from jax.experimental.pallas import tpu_sc as plsc

NUM_WORKERS = 32
CHUNK = 16
NBUF = 6
AHEAD = 6


def _make_gather(nrows, ncols, embed):
    total = nrows * ncols
    per_worker = total // NUM_WORKERS
    nchunks = per_worker // CHUNK
    wpr = ncols // per_worker
    mesh = plsc.VectorSubcoreMesh(core_axis_name="c", subcore_axis_name="s")

    @functools.partial(
        pl.kernel,
        mesh=mesh,
        out_type=jax.ShapeDtypeStruct((nrows, ncols, embed), jnp.float32),
        scratch_types=[
            pltpu.VMEM((per_worker,), jnp.int32),
        ] + [pltpu.VMEM((CHUNK, embed), jnp.float32)] * NBUF
          + [pltpu.SemaphoreType.DMA] * (2 * NBUF),
    )
    def gather_kernel(idx_hbm, table_hbm, out_hbm, idx_v, *scratch):
        bufs = scratch[:NBUF]
        gsems = scratch[NBUF:2 * NBUF]
        wsems = scratch[2 * NBUF:]
        wid = lax.axis_index("c") * (NUM_WORKERS // 2) + lax.axis_index("s")
        row = wid // wpr
        col = (wid % wpr) * per_worker
        pltpu.sync_copy(idx_hbm.at[row, pl.ds(col, per_worker)], idx_v)

        def start_gather(j):
            return pltpu.async_copy(
                table_hbm.at[idx_v.at[pl.ds(j * CHUNK, CHUNK)]],
                bufs[j % NBUF], gsems[j % NBUF])

        gds = [None] * nchunks
        wds = [None] * nchunks
        waited = [False] * nchunks
        for j in range(min(AHEAD, nchunks)):
            gds[j] = start_gather(j)
        for j in range(nchunks):
            gds[j].wait()
            wds[j] = pltpu.async_copy(
                bufs[j % NBUF],
                out_hbm.at[row, pl.ds(col + j * CHUNK, CHUNK)],
                wsems[j % NBUF])
            nxt = j + AHEAD
            if nxt < nchunks:
                if nxt >= NBUF:
                    wds[nxt - NBUF].wait()
                    waited[nxt - NBUF] = True
                gds[nxt] = start_gather(nxt)
        for j in range(nchunks):
            if not waited[j]:
                wds[j].wait()

    return gather_kernel


def kernel(time, pos_enc):
    if time.dtype != jnp.int32:
        time = time.astype(jnp.int32)
    nrows, ncols = time.shape
    return _make_gather(nrows, ncols, pos_enc.shape[1])(time, pos_enc)
